# Initial kernel scaffold; baseline (speedup 1.0000x reference)
import jax
import jax.numpy as jnp
from jax.experimental import pallas as pl


def kernel(sub0_x, sub1_x, structural_features, sub0_node_ids, sub1_node_ids, sub0_edge_index, sub1_edge_index, struct_edge_index, c0_W0, c0_b0, c0_W1, c0_b1, c0_W2, c0_b2, c1_W0, c1_b0, c1_W1, c1_b1, c1_W2, c1_b2, s_W0, s_b0, s_W1, s_b1):
    raise NotImplementedError("write your pallas kernel here")


# speedup vs baseline: 6.9124x; 6.9124x over previous
import functools

import jax
import jax.numpy as jnp
from jax import lax
from jax.experimental import pallas as pl
from jax.experimental.pallas import tpu as pltpu
from jax.experimental.pallas import tpu_sc as plsc

F32 = jnp.float32

_N_SUB = 5000
_N_STRUCT = 10000
_D = 128
_NCLS = 64

_NZ_C = 5120
_NZ_S = 10240

_CH = 128


def _pad_rows(x, nz):
    return jnp.pad(x, ((0, nz - x.shape[0]), (0, 0)))


@functools.partial(jax.jit, static_argnums=(1,))
def _deg_partials(dst_pad, np_acc):
    e_pad = dst_pad.shape[0]
    per_w = e_pad // 32
    cw = np_acc // 16

    mesh = plsc.VectorSubcoreMesh(core_axis_name="c", subcore_axis_name="s")

    @functools.partial(
        pl.kernel,
        out_type=jax.ShapeDtypeStruct((2 * np_acc,), F32),
        mesh=mesh,
        scratch_types=[
            pltpu.VMEM((per_w,), jnp.int32),
            pltpu.VMEM((np_acc,), F32),
            pltpu.VMEM((16 * cw,), F32),
            pltpu.VMEM((cw,), F32),
            pltpu.VMEM_SHARED((16 * np_acc,), F32),
        ],
        compiler_params=pltpu.CompilerParams(needs_layout_passes=False),
    )
    def k(dst_hbm, out_hbm, idx_v, acc_v, part_v, res_v, sh):
        c = lax.axis_index("c")
        s = lax.axis_index("s")
        w = s * 2 + c
        zero16 = jnp.zeros((16,), F32)
        one16 = jnp.full((16,), 1.0, F32)

        def zbody(i, carry):
            acc_v[pl.ds(i * 16, 16)] = zero16
            return carry
        lax.fori_loop(0, np_acc // 16, zbody, 0)

        pltpu.sync_copy(dst_hbm.at[pl.ds(w * per_w, per_w)], idx_v)

        def cbody(i, carry):
            idx = idx_v[pl.ds(i * 16, 16)]
            plsc.addupdate_scatter(acc_v, [idx], one16)
            return carry
        lax.fori_loop(0, per_w // 16, cbody, 0)

        pltpu.sync_copy(acc_v, sh.at[pl.ds(s * np_acc, np_acc)])
        plsc.subcore_barrier()

        for r in range(16):
            pltpu.sync_copy(sh.at[pl.ds(r * np_acc + s * cw, cw)],
                            part_v.at[pl.ds(r * cw, cw)])
        for j in range(cw // 16):
            v = part_v[pl.ds(j * 16, 16)]
            for r in range(1, 16):
                v = v + part_v[pl.ds(r * cw + j * 16, 16)]
            res_v[pl.ds(j * 16, 16)] = v
        pltpu.sync_copy(res_v, out_hbm.at[pl.ds(c * np_acc + s * cw, cw)])

    return k(dst_pad).reshape(2, np_acc)


@functools.partial(jax.jit, static_argnums=())
def _sc_gather(table, ids):
    g = ids.shape[0]
    per_w = g // 32
    n_ch = per_w // 32

    mesh = plsc.VectorSubcoreMesh(core_axis_name="c", subcore_axis_name="s")

    @functools.partial(
        pl.kernel,
        out_type=jax.ShapeDtypeStruct((g, _D), F32),
        mesh=mesh,
        scratch_types=[
            pltpu.VMEM((per_w,), jnp.int32),
            pltpu.VMEM((32, _D), F32),
            pltpu.SemaphoreType.DMA,
        ],
        compiler_params=pltpu.CompilerParams(needs_layout_passes=False),
    )
    def k(table_hbm, ids_hbm, out_hbm, idx_v, rows_v, sem):
        c = lax.axis_index("c")
        s = lax.axis_index("s")
        w = s * 2 + c
        pltpu.sync_copy(ids_hbm.at[pl.ds(w * per_w, per_w)], idx_v)

        def body(j, carry):
            pltpu.async_copy(table_hbm.at[idx_v.at[pl.ds(j * 32, 32)]],
                             rows_v, sem).wait()
            pltpu.sync_copy(rows_v, out_hbm.at[pl.ds(w * per_w + j * 32, 32)])
            return carry
        lax.fori_loop(0, n_ch, body, 0)

    return k(table, ids)


@functools.partial(jax.jit, static_argnums=())
def _sc_edge_pass(y, src1d, dst1d):
    nz = y.shape[0]
    e_pad = src1d.shape[0]
    per_w = e_pad // 32
    n_chunks = per_w // _CH
    rows_pt = nz // 16

    mesh = plsc.VectorSubcoreMesh(core_axis_name="c", subcore_axis_name="s")

    @functools.partial(
        pl.kernel,
        out_type=jax.ShapeDtypeStruct((2, nz, _D), F32),
        mesh=mesh,
        scratch_types=[
            pltpu.VMEM((_CH,), jnp.int32),
            pltpu.VMEM((_CH,), jnp.int32),
            pltpu.VMEM((_CH, _D), F32),
            pltpu.VMEM_SHARED((nz, _D), F32),
            pltpu.SemaphoreType.DMA,
        ],
        compiler_params=pltpu.CompilerParams(needs_layout_passes=False),
    )
    def k(y_hbm, src_hbm, dst_hbm, out_hbm, src_c, dst_c, rows_v, z_sh, sem):
        c = lax.axis_index("c")
        s = lax.axis_index("s")
        w = s * 2 + c
        ebase = w * per_w
        rbase = s * rows_pt
        pltpu.sync_copy(y_hbm.at[pl.ds(rbase, rows_pt)],
                        z_sh.at[pl.ds(rbase, rows_pt)])
        plsc.subcore_barrier()

        def body(j, carry):
            pltpu.sync_copy(src_hbm.at[pl.ds(ebase + j * _CH, _CH)], src_c)
            pltpu.sync_copy(dst_hbm.at[pl.ds(ebase + j * _CH, _CH)], dst_c)
            pltpu.async_copy(y_hbm.at[src_c], rows_v, sem).wait()
            pltpu.sync_copy(rows_v, z_sh.at[dst_c], add=True)
            return carry
        lax.fori_loop(0, n_chunks, body, 0)

        plsc.subcore_barrier()
        pltpu.sync_copy(z_sh.at[pl.ds(rbase, rows_pt)],
                        out_hbm.at[c].at[pl.ds(rbase, rows_pt)])

    return k(y, src1d, dst1d)


def _tc_pre(h, sg, dinv2d, w):
    nz = h.shape[0]
    dout = w.shape[1]
    bm = 512
    nb = nz // bm

    def body(h_ref, sg_ref, dv_ref, w_ref, o_ref):
        wb = w_ref[...]
        acc = jnp.dot(h_ref[...], wb[:_D], preferred_element_type=F32)
        acc = acc + jnp.dot(sg_ref[...], wb[_D:], preferred_element_type=F32)
        o_ref[...] = dv_ref[...] * acc

    return pl.pallas_call(
        body,
        grid=(nb,),
        in_specs=[
            pl.BlockSpec((bm, _D), lambda i: (i, 0)),
            pl.BlockSpec((bm, _D), lambda i: (i, 0)),
            pl.BlockSpec((bm, _D), lambda i: (i, 0)),
            pl.BlockSpec((2 * _D, dout), lambda i: (0, 0)),
        ],
        out_specs=pl.BlockSpec((bm, dout), lambda i: (i, 0)),
        out_shape=jax.ShapeDtypeStruct((nz, dout), F32),
    )(h, sg, dinv2d, w)


def _tc_pre_struct(sp, dinv2d, w):
    nz = sp.shape[0]
    bm = 512
    nb = nz // bm

    def body(s_ref, dv_ref, w_ref, o_ref):
        acc = jnp.dot(s_ref[...], w_ref[...], preferred_element_type=F32)
        o_ref[...] = dv_ref[...] * acc

    return pl.pallas_call(
        body,
        grid=(nb,),
        in_specs=[
            pl.BlockSpec((bm, _D), lambda i: (i, 0)),
            pl.BlockSpec((bm, _D), lambda i: (i, 0)),
            pl.BlockSpec((_D, _D), lambda i: (0, 0)),
        ],
        out_specs=pl.BlockSpec((bm, _D), lambda i: (i, 0)),
        out_shape=jax.ShapeDtypeStruct((nz, _D), F32),
    )(sp, dinv2d, w)


def _tc_post(z, y, dinv2d, b2d, mode):
    nz = y.shape[0]
    bm = 512
    nb = nz // bm
    dout = _NCLS if mode == "softmax" else _D

    def body(z_ref, y_ref, dv_ref, b_ref, o_ref):
        t = z_ref[0] + z_ref[1] - y_ref[...]
        hh = dv_ref[...] * t + b_ref[...]
        if mode == "relu":
            o_ref[...] = jnp.maximum(hh, 0.0)
        elif mode == "none":
            o_ref[...] = hh
        else:
            l = hh[:, :_NCLS]
            m = jnp.max(l, axis=1, keepdims=True)
            e = jnp.exp(l - m)
            o_ref[...] = e / jnp.sum(e, axis=1, keepdims=True)

    return pl.pallas_call(
        body,
        grid=(nb,),
        in_specs=[
            pl.BlockSpec((2, bm, _D), lambda i: (0, i, 0)),
            pl.BlockSpec((bm, _D), lambda i: (i, 0)),
            pl.BlockSpec((bm, _D), lambda i: (i, 0)),
            pl.BlockSpec((1, _D), lambda i: (0, 0)),
        ],
        out_specs=pl.BlockSpec((bm, dout), lambda i: (i, 0)),
        out_shape=jax.ShapeDtypeStruct((nz, dout), F32),
    )(z, y, dinv2d, b2d)


def _prep_edges(edge_index, n):
    src, dst = edge_index[0], edge_index[1]
    e = src.shape[0]
    e_pad = ((e + 32 * _CH - 1) // (32 * _CH)) * (32 * _CH)
    src_p = jnp.pad(src.astype(jnp.int32), (0, e_pad - e))
    dst_p = jnp.pad(dst.astype(jnp.int32), (0, e_pad - e),
                    constant_values=jnp.int32(n))
    return (src_p, dst_p)


def _dinv2d_of(deg_partials_2xn, n, nz):
    deg = deg_partials_2xn[0, :n] + deg_partials_2xn[1, :n] + 1.0
    dinv = lax.rsqrt(deg)
    dinv = jnp.pad(dinv, (0, nz - n), constant_values=1.0)
    return jnp.broadcast_to(dinv[:, None], (nz, _D))


def _pad_w(w, b):
    wp = jnp.pad(w, ((0, 0), (0, _D - w.shape[1])))
    bp = jnp.pad(b, (0, _D - b.shape[0]))
    return wp, bp


def kernel(sub0_x, sub1_x, structural_features, sub0_node_ids, sub1_node_ids,
           sub0_edge_index, sub1_edge_index, struct_edge_index,
           c0_W0, c0_b0, c0_W1, c0_b1, c0_W2, c0_b2,
           c1_W0, c1_b0, c1_W1, c1_b1, c1_W2, c1_b2,
           s_W0, s_b0, s_W1, s_b1):
    c0_W2, c0_b2 = _pad_w(c0_W2, c0_b2)
    c1_W2, c1_b2 = _pad_w(c1_W2, c1_b2)
    cw = [[(c0_W0, c0_b0), (c0_W1, c0_b1), (c0_W2, c0_b2)],
          [(c1_W0, c1_b0), (c1_W1, c1_b1), (c1_W2, c1_b2)]]
    sw = [(s_W0, s_b0), (s_W1, s_b1)]

    h = [_pad_rows(sub0_x, _NZ_C), _pad_rows(sub1_x, _NZ_C)]
    sp = _pad_rows(structural_features, _NZ_S)
    ids_p = [
        jnp.pad(sub0_node_ids.astype(jnp.int32), (0, _NZ_C - _N_SUB)),
        jnp.pad(sub1_node_ids.astype(jnp.int32), (0, _NZ_C - _N_SUB)),
    ]
    edges = [_prep_edges(sub0_edge_index, _N_SUB),
             _prep_edges(sub1_edge_index, _N_SUB),
             _prep_edges(struct_edge_index, _N_STRUCT)]

    dinv2d = []
    for gi, nn, nz in ((0, _N_SUB, _NZ_C), (1, _N_SUB, _NZ_C),
                       (2, _N_STRUCT, _NZ_S)):
        dp = _deg_partials(edges[gi][1], nz)
        dinv2d.append(_dinv2d_of(dp, nn, nz))

    for layer in range(2):
        new_h = []
        for cid in range(2):
            sg = _sc_gather(sp, ids_p[cid])
            w, b = cw[cid][layer]
            y = _tc_pre(h[cid], sg, dinv2d[cid], w)
            z = _sc_edge_pass(y, edges[cid][0], edges[cid][1])
            new_h.append(_tc_post(z, y, dinv2d[cid], b.reshape(1, -1), "relu"))
        ws, bs = sw[layer]
        ys = _tc_pre_struct(sp, dinv2d[2], ws)
        zs = _sc_edge_pass(ys, edges[2][0], edges[2][1])
        sp = _tc_post(zs, ys, dinv2d[2], bs.reshape(1, -1),
                      "relu" if layer == 0 else "none")
        h = new_h

    outs = []
    for cid in range(2):
        sg = _sc_gather(sp, ids_p[cid])
        w, b = cw[cid][2]
        y = _tc_pre(h[cid], sg, dinv2d[cid], w)
        z = _sc_edge_pass(y, edges[cid][0], edges[cid][1])
        p = _tc_post(z, y, dinv2d[cid], b.reshape(1, -1), "softmax")
        outs.append(p[:_N_SUB])

    return (sp[:_N_STRUCT], outs[0], outs[1])
